# Initial kernel scaffold; baseline (speedup 1.0000x reference)
import jax
import jax.numpy as jnp
from jax.experimental import pallas as pl


def kernel(offset_imgs, vt_idx_front, vt_idx_back, mask_front, mask_back):
    raise NotImplementedError("write your pallas kernel here")


# speedup vs baseline: 1.2394x; 1.2394x over previous
import jax
import jax.numpy as jnp
from jax import lax
from jax.experimental import pallas as pl
from jax.experimental.pallas import tpu as pltpu
from jax.experimental.pallas import tpu_sc as plsc

B = 16
C = 6
HW = 512 * 512
NV = 100000
NC = 2
NS = 16
NW = NC * NS
CH = 3136
NVP = NW * CH
G = CH // 16
G3 = 3 * G


def _div3(x):
    return lax.shift_right_logical(x * 43691, 17)


def _body(img, idxf, idxb, mf, mb, out,
          idxf_v, idxb_v, ixf3_v, ixb3_v, wf_v, wb_v, wf3_v, wb3_v,
          gf3_v, gb3_v, out_v, sem):
    cid = lax.axis_index("c")
    sid = lax.axis_index("s")
    wid = sid * NC + cid
    base = wid * CH

    pltpu.sync_copy(idxf.at[pl.ds(base, CH)], idxf_v)
    pltpu.sync_copy(idxb.at[pl.ds(base, CH)], idxb_v)
    pltpu.sync_copy(mf.at[pl.ds(base, CH)], ixf3_v.at[pl.ds(0, CH)])
    pltpu.sync_copy(mb.at[pl.ds(base, CH)], ixb3_v.at[pl.ds(0, CH)])

    def wloop(i, carry):
        s = pl.ds(i * 16, 16)
        a = lax.convert_element_type(ixf3_v[s], jnp.float32)
        bb = lax.convert_element_type(ixb3_v[s], jnp.float32)
        d = jnp.maximum(a + bb, 1.0)
        wf_v[s] = a / d
        wb_v[s] = bb / d
        return carry

    lax.fori_loop(0, G, wloop, 0)

    def eloop(k, carry):
        vb = k * 16
        wsrc = wf_v[pl.ds(vb, 16)]
        vsrc = wb_v[pl.ds(vb, 16)]
        fsrc = idxf_v[pl.ds(vb, 16)]
        bsrc = idxb_v[pl.ds(vb, 16)]
        for j in range(3):
            s = pl.ds(k * 48 + j * 16, 16)
            io = lax.iota(jnp.int32, 16) + (j * 16)
            v = _div3(io)
            cvec = io - v * 3
            wf3_v[s] = wsrc.at[v].get(mode="promise_in_bounds")
            wb3_v[s] = vsrc.at[v].get(mode="promise_in_bounds")
            ixf3_v[s] = fsrc.at[v].get(mode="promise_in_bounds") + cvec * HW
            ixb3_v[s] = bsrc.at[v].get(mode="promise_in_bounds") + (cvec + 3) * HW
        return carry

    lax.fori_loop(0, G, eloop, 0)

    def bloop(b, carry):
        cp_f = pltpu.async_copy(img.at[ixf3_v], gf3_v, sem)
        cp_b = pltpu.async_copy(img.at[ixb3_v], gb3_v, sem)
        cp_f.wait()
        cp_b.wait()

        def mloop(i, cc):
            s = pl.ds(i * 16, 16)
            out_v[s] = gf3_v[s] * wf3_v[s] + gb3_v[s] * wb3_v[s]
            ixf3_v[s] = ixf3_v[s] + (C * HW)
            ixb3_v[s] = ixb3_v[s] + (C * HW)
            return cc

        lax.fori_loop(0, G3, mloop, 0)

        pltpu.sync_copy(out_v, out.at[pl.ds(b * (NVP * 3) + base * 3, CH * 3)])
        return carry

    lax.fori_loop(0, B, bloop, 0)


def _make_sample():
    return pl.kernel(
        _body,
        mesh=plsc.VectorSubcoreMesh(core_axis_name="c", subcore_axis_name="s"),
        out_type=jax.ShapeDtypeStruct((B * NVP * 3,), jnp.float32),
        scratch_types=[
            pltpu.VMEM((CH,), jnp.int32),
            pltpu.VMEM((CH,), jnp.int32),
            pltpu.VMEM((3 * CH,), jnp.int32),
            pltpu.VMEM((3 * CH,), jnp.int32),
            pltpu.VMEM((CH,), jnp.float32),
            pltpu.VMEM((CH,), jnp.float32),
            pltpu.VMEM((3 * CH,), jnp.float32),
            pltpu.VMEM((3 * CH,), jnp.float32),
            pltpu.VMEM((3 * CH,), jnp.float32),
            pltpu.VMEM((3 * CH,), jnp.float32),
            pltpu.VMEM((3 * CH,), jnp.float32),
            pltpu.SemaphoreType.DMA,
        ],
    )


def kernel(offset_imgs, vt_idx_front, vt_idx_back, mask_front, mask_back):
    img = offset_imgs.reshape(B * C * HW)
    pad = NVP - NV
    idxf = jnp.pad(vt_idx_front, (0, pad))
    idxb = jnp.pad(vt_idx_back, (0, pad))
    mf = jnp.pad(mask_front, (0, pad))
    mb = jnp.pad(mask_back, (0, pad))
    o = _make_sample()(img, idxf, idxb, mf, mb)
    return o.reshape(B, NVP, 3)[:, :NV, :]
